# Initial kernel scaffold; baseline (speedup 1.0000x reference)
import functools

import jax
import jax.numpy as jnp
from jax import lax
from jax.experimental import pallas as pl
from jax.experimental.pallas import tpu as pltpu
from jax.experimental.pallas import tpu_sc as plsc

N = 10000
NPAD = 10240
NC, NS = 2, 16
B = 128
T = 162
TE = 81
E_PAD = NS * T * B
RPT = NPAD // NS
BR = 1024

_MESH = plsc.VectorSubcoreMesh(core_axis_name="c", subcore_axis_name="s")


@functools.partial(
    pl.kernel,
    out_type=jax.ShapeDtypeStruct((NC, NPAD), jnp.float32),
    mesh=_MESH,
    scratch_types=[
        pltpu.VMEM((TE, B), jnp.int32),
        pltpu.VMEM((B,), jnp.float32),
        pltpu.VMEM_SHARED((NPAD,), jnp.float32),
    ],
)
def _deg_kernel(dst_hbm, zeros_hbm, ones_hbm, out, idx_v, ones_v, acc):
    c = lax.axis_index("c")
    s = lax.axis_index("s")
    wid = c * NS + s
    pltpu.sync_copy(zeros_hbm, acc.at[pl.ds(s * RPT, RPT)])
    pltpu.sync_copy(ones_hbm, ones_v)
    pltpu.sync_copy(dst_hbm.at[wid], idx_v)
    plsc.subcore_barrier()

    @pl.loop(0, TE)
    def _(j):
        pltpu.sync_copy(ones_v, acc.at[idx_v.at[j]], add=True)

    plsc.subcore_barrier()
    pltpu.sync_copy(acc.at[pl.ds(s * RPT, RPT)], out.at[c, pl.ds(s * RPT, RPT)])


def _make_scatter(CW):
    @functools.partial(
        pl.kernel,
        out_type=jax.ShapeDtypeStruct((NC, NPAD, CW), jnp.float32),
        mesh=_MESH,
        scratch_types=[
            pltpu.VMEM((T, B), jnp.int32),
            pltpu.VMEM((T, B), jnp.int32),
            pltpu.VMEM((B, CW), jnp.float32),
            pltpu.VMEM((B, CW), jnp.float32),
            pltpu.SemaphoreType.DMA,
            pltpu.SemaphoreType.DMA,
            pltpu.VMEM_SHARED((NPAD, CW), jnp.float32),
        ],
    )
    def _scatter(g_hbm, src_hbm, dst_hbm, zeros_hbm, out,
                 sidx, didx, buf0, buf1, sem0, sem1, acc):
        c = lax.axis_index("c")
        s = lax.axis_index("s")
        pltpu.sync_copy(zeros_hbm, acc.at[pl.ds(s * RPT, RPT)])
        pltpu.sync_copy(src_hbm.at[c, s], sidx)
        pltpu.sync_copy(dst_hbm.at[s], didx)
        plsc.subcore_barrier()

        pltpu.async_copy(g_hbm.at[sidx.at[0]], buf0, sem0)
        pltpu.async_copy(g_hbm.at[sidx.at[1]], buf1, sem1)

        @pl.loop(0, T, step=2)
        def _(j):
            pltpu.make_async_copy(g_hbm.at[sidx.at[j]], buf0, sem0).wait()
            pltpu.sync_copy(buf0, acc.at[didx.at[j]], add=True)

            @pl.when(j + 2 < T)
            def _():
                pltpu.async_copy(g_hbm.at[sidx.at[j + 2]], buf0, sem0)

            pltpu.make_async_copy(g_hbm.at[sidx.at[j + 1]], buf1, sem1).wait()
            pltpu.sync_copy(buf1, acc.at[didx.at[j + 1]], add=True)

            @pl.when(j + 3 < T)
            def _():
                pltpu.async_copy(g_hbm.at[sidx.at[j + 3]], buf1, sem1)

        plsc.subcore_barrier()
        pltpu.sync_copy(acc.at[pl.ds(s * RPT, RPT)],
                        out.at[c, pl.ds(s * RPT, RPT), :])

    return _scatter


_scatter128 = _make_scatter(128)
_scatter64 = _make_scatter(64)


def _k1_body(h_ref, d0_ref, d1_ref, w_ref, dis_ref, g_ref):
    deg = d0_ref[...] + d1_ref[...]
    dis = jnp.where(deg > 0.0, lax.rsqrt(deg), 0.0)
    dis_ref[...] = dis
    hw = jnp.dot(h_ref[...], w_ref[...],
                 preferred_element_type=jnp.float32,
                 precision=lax.Precision.HIGHEST) * dis
    g_ref[0, :, :] = hw[:, :128]
    g_ref[1, :, :] = hw[:, 128:]


def _k1(h, d0, d1, W1):
    grid = NPAD // BR
    return pl.pallas_call(
        _k1_body,
        grid=(grid,),
        in_specs=[
            pl.BlockSpec((BR, 128), lambda i: (i, 0)),
            pl.BlockSpec((BR, 1), lambda i: (i, 0)),
            pl.BlockSpec((BR, 1), lambda i: (i, 0)),
            pl.BlockSpec((128, 256), lambda i: (0, 0)),
        ],
        out_specs=[
            pl.BlockSpec((BR, 1), lambda i: (i, 0)),
            pl.BlockSpec((NC, BR, 128), lambda i: (0, i, 0)),
        ],
        out_shape=[
            jax.ShapeDtypeStruct((NPAD, 1), jnp.float32),
            jax.ShapeDtypeStruct((NC, NPAD, 128), jnp.float32),
        ],
    )(h, d0, d1, W1)


def _mk_mid_body(CWO):
    def body(a_ref, dis_ref, b_ref, w_ref, g_ref):
        dis = dis_ref[...]
        s0 = jax.nn.relu(a_ref[0, :, :] * dis + b_ref[:, :128])
        s1 = jax.nn.relu(a_ref[1, :, :] * dis + b_ref[:, 128:])
        hw = (jnp.dot(s0, w_ref[:128, :],
                      preferred_element_type=jnp.float32,
                      precision=lax.Precision.HIGHEST)
              + jnp.dot(s1, w_ref[128:, :],
                        preferred_element_type=jnp.float32,
                        precision=lax.Precision.HIGHEST)) * dis
        g_ref[0, :, :] = hw[:, :CWO]
        g_ref[1, :, :] = hw[:, CWO:]
    return body


def _mid(a, dis, b, W, CWO):
    grid = NPAD // BR
    dout = 2 * CWO
    return pl.pallas_call(
        _mk_mid_body(CWO),
        grid=(grid,),
        in_specs=[
            pl.BlockSpec((NC, BR, 128), lambda i: (0, i, 0)),
            pl.BlockSpec((BR, 1), lambda i: (i, 0)),
            pl.BlockSpec((1, 256), lambda i: (0, 0)),
            pl.BlockSpec((256, dout), lambda i: (0, 0)),
        ],
        out_specs=pl.BlockSpec((NC, BR, CWO), lambda i: (0, i, 0)),
        out_shape=jax.ShapeDtypeStruct((NC, NPAD, CWO), jnp.float32),
    )(a, dis, b, W)


def _k4_body(a_ref, dis_ref, b_ref, o_ref):
    a = jnp.concatenate([a_ref[0, :, :], a_ref[1, :, :]], axis=1)
    o_ref[...] = jax.nn.relu(a * dis_ref[...] + b_ref[...])


def _k4(a, dis, b3):
    grid = NPAD // BR
    return pl.pallas_call(
        _k4_body,
        grid=(grid,),
        in_specs=[
            pl.BlockSpec((NC, BR, 64), lambda i: (0, i, 0)),
            pl.BlockSpec((BR, 1), lambda i: (i, 0)),
            pl.BlockSpec((1, 128), lambda i: (0, 0)),
        ],
        out_specs=pl.BlockSpec((BR, 128), lambda i: (i, 0)),
        out_shape=jax.ShapeDtypeStruct((NPAD, 128), jnp.float32),
    )(a, dis, b3)


def kernel(x, batch_edge_index, W1, b1, W2, b2, W3, b3):
    n_batch, n_points, _ = x.shape
    h0 = jnp.pad(x.reshape(-1, x.shape[-1]), ((0, NPAD - N), (0, 0)))

    loop = jnp.arange(N, dtype=jnp.int32)
    npad_e = E_PAD - (batch_edge_index.shape[1] + N)
    pad_src = (jnp.arange(npad_e, dtype=jnp.int32) * 181) % N
    pad_dst = N + (jnp.arange(npad_e, dtype=jnp.int32) % (NPAD - N))
    srcp = jnp.concatenate([batch_edge_index[0], loop, pad_src])
    dstp = jnp.concatenate([batch_edge_index[1], loop, pad_dst])

    src_col = srcp.reshape(NS, T, B)
    src_off = jnp.stack([src_col, src_col + NPAD])
    dst_col = dstp.reshape(NS, T, B)
    dst_edge = dstp.reshape(NC * NS, TE, B)

    zeros_r = jnp.zeros((RPT,), jnp.float32)
    ones_b = jnp.ones((B,), jnp.float32)
    zeros128 = jnp.zeros((RPT, 128), jnp.float32)
    zeros64 = jnp.zeros((RPT, 64), jnp.float32)

    degs = _deg_kernel(dst_edge, zeros_r, ones_b)
    d0 = degs[0].reshape(NPAD, 1)
    d1 = degs[1].reshape(NPAD, 1)

    dis, g1 = _k1(h0, d0, d1, W1)
    a1 = _scatter128(g1.reshape(NC * NPAD, 128), src_off, dst_col, zeros128)
    g2 = _mid(a1, dis, b1.reshape(1, 256), W2, 128)
    a2 = _scatter128(g2.reshape(NC * NPAD, 128), src_off, dst_col, zeros128)
    g3 = _mid(a2, dis, b2.reshape(1, 256), W3, 64)
    a3 = _scatter64(g3.reshape(NC * NPAD, 64), src_off, dst_col, zeros64)
    out = _k4(a3, dis, b3.reshape(1, 128))
    return out[:N].reshape(n_batch, n_points, -1)


# speedup vs baseline: 14.8950x; 14.8950x over previous
import functools

import jax
import jax.numpy as jnp
from jax import lax
from jax.experimental import pallas as pl
from jax.experimental.pallas import tpu as pltpu
from jax.experimental.pallas import tpu_sc as plsc

N = 10000
NPAD = 10240
NC, NS = 2, 16
B = 128
T = 168
CT = 56
TE = 96
CTE = 48
E_COL = NS * T * B
E_EDG = NC * NS * TE * B
RPT = NPAD // NS
BR = 1024

_MESH = plsc.VectorSubcoreMesh(core_axis_name="c", subcore_axis_name="s")


def _pipeline_chunk(g_hbm, acc, sidx, didx, buf0, buf1, sem0, sem1, n):
    pltpu.async_copy(g_hbm.at[sidx.at[0]], buf0, sem0)
    pltpu.async_copy(g_hbm.at[sidx.at[1]], buf1, sem1)

    @pl.loop(0, n, step=2)
    def _(j):
        pltpu.make_async_copy(g_hbm.at[sidx.at[j]], buf0, sem0).wait()
        pltpu.sync_copy(buf0, acc.at[didx.at[j]], add=True)

        @pl.when(j + 2 < n)
        def _():
            pltpu.async_copy(g_hbm.at[sidx.at[j + 2]], buf0, sem0)

        pltpu.make_async_copy(g_hbm.at[sidx.at[j + 1]], buf1, sem1).wait()
        pltpu.sync_copy(buf1, acc.at[didx.at[j + 1]], add=True)

        @pl.when(j + 3 < n)
        def _():
            pltpu.async_copy(g_hbm.at[sidx.at[j + 3]], buf1, sem1)


@functools.partial(
    pl.kernel,
    out_type=jax.ShapeDtypeStruct((NC, NPAD, 128), jnp.float32),
    mesh=_MESH,
    scratch_types=[
        pltpu.VMEM((CT, B), jnp.int32),
        pltpu.VMEM((CT, B), jnp.int32),
        pltpu.VMEM((B, 128), jnp.float32),
        pltpu.VMEM((B, 128), jnp.float32),
        pltpu.SemaphoreType.DMA,
        pltpu.SemaphoreType.DMA,
        pltpu.VMEM_SHARED((NPAD, 128), jnp.float32),
    ],
)
def _scatter_col(g_hbm, src_hbm, dst_hbm, zeros_hbm, out,
                 sidx, didx, buf0, buf1, sem0, sem1, acc):
    c = lax.axis_index("c")
    s = lax.axis_index("s")
    pltpu.sync_copy(zeros_hbm, acc.at[pl.ds(s * RPT, RPT)])
    plsc.subcore_barrier()

    for k in range(T // CT):
        pltpu.sync_copy(src_hbm.at[c, s, pl.ds(k * CT, CT)], sidx)
        pltpu.sync_copy(dst_hbm.at[s, pl.ds(k * CT, CT)], didx)
        _pipeline_chunk(g_hbm, acc, sidx, didx, buf0, buf1, sem0, sem1, CT)

    plsc.subcore_barrier()
    pltpu.sync_copy(acc.at[pl.ds(s * RPT, RPT)],
                    out.at[c, pl.ds(s * RPT, RPT), :])


@functools.partial(
    pl.kernel,
    out_type=jax.ShapeDtypeStruct((NC, NPAD, 128), jnp.float32),
    mesh=_MESH,
    scratch_types=[
        pltpu.VMEM((CTE, B), jnp.int32),
        pltpu.VMEM((CTE, B), jnp.int32),
        pltpu.VMEM((B, 128), jnp.float32),
        pltpu.VMEM((B, 128), jnp.float32),
        pltpu.SemaphoreType.DMA,
        pltpu.SemaphoreType.DMA,
        pltpu.VMEM_SHARED((NPAD, 128), jnp.float32),
    ],
)
def _scatter_edge(g_hbm, src_hbm, dst_hbm, zeros_hbm, out,
                  sidx, didx, buf0, buf1, sem0, sem1, acc):
    c = lax.axis_index("c")
    s = lax.axis_index("s")
    wid = c * NS + s
    pltpu.sync_copy(zeros_hbm, acc.at[pl.ds(s * RPT, RPT)])
    plsc.subcore_barrier()

    for k in range(TE // CTE):
        pltpu.sync_copy(src_hbm.at[wid, pl.ds(k * CTE, CTE)], sidx)
        pltpu.sync_copy(dst_hbm.at[wid, pl.ds(k * CTE, CTE)], didx)
        _pipeline_chunk(g_hbm, acc, sidx, didx, buf0, buf1, sem0, sem1, CTE)

    plsc.subcore_barrier()
    pltpu.sync_copy(acc.at[pl.ds(s * RPT, RPT)],
                    out.at[c, pl.ds(s * RPT, RPT), :])


def _k1_body(h_ref, deg_ref, w_ref, dis_ref, g_ref):
    deg = deg_ref[0, :, 0:1]
    dis = jnp.where(deg > 0.0, lax.rsqrt(deg), 0.0)
    dis_ref[...] = dis
    hw = jnp.dot(h_ref[...], w_ref[...],
                 preferred_element_type=jnp.float32,
                 precision=lax.Precision.HIGHEST) * dis
    g_ref[0, :, :] = hw[:, :128]
    g_ref[1, :, :] = hw[:, 128:]


def _k1(h, deg_full, W1):
    grid = NPAD // BR
    return pl.pallas_call(
        _k1_body,
        grid=(grid,),
        in_specs=[
            pl.BlockSpec((BR, 128), lambda i: (i, 0)),
            pl.BlockSpec((1, BR, 128), lambda i: (0, i, 0)),
            pl.BlockSpec((128, 256), lambda i: (0, 0)),
        ],
        out_specs=[
            pl.BlockSpec((BR, 1), lambda i: (i, 0)),
            pl.BlockSpec((NC, BR, 128), lambda i: (0, i, 0)),
        ],
        out_shape=[
            jax.ShapeDtypeStruct((NPAD, 1), jnp.float32),
            jax.ShapeDtypeStruct((NC, NPAD, 128), jnp.float32),
        ],
    )(h, deg_full, W1)


def _mk_mid_body(split_out):
    def body(a_ref, dis_ref, b_ref, w_ref, g_ref):
        dis = dis_ref[...]
        s0 = jax.nn.relu(a_ref[0, :, :] * dis + b_ref[:, :128])
        s1 = jax.nn.relu(a_ref[1, :, :] * dis + b_ref[:, 128:])
        hw = (jnp.dot(s0, w_ref[:128, :],
                      preferred_element_type=jnp.float32,
                      precision=lax.Precision.HIGHEST)
              + jnp.dot(s1, w_ref[128:, :],
                        preferred_element_type=jnp.float32,
                        precision=lax.Precision.HIGHEST)) * dis
        if split_out:
            g_ref[0, :, :] = hw[:, :128]
            g_ref[1, :, :] = hw[:, 128:]
        else:
            g_ref[...] = hw
    return body


def _mid(a, dis, b, W, split_out):
    grid = NPAD // BR
    dout = W.shape[1]
    if split_out:
        out_spec = pl.BlockSpec((NC, BR, 128), lambda i: (0, i, 0))
        out_shape = jax.ShapeDtypeStruct((NC, NPAD, 128), jnp.float32)
    else:
        out_spec = pl.BlockSpec((BR, dout), lambda i: (i, 0))
        out_shape = jax.ShapeDtypeStruct((NPAD, dout), jnp.float32)
    return pl.pallas_call(
        _mk_mid_body(split_out),
        grid=(grid,),
        in_specs=[
            pl.BlockSpec((NC, BR, 128), lambda i: (0, i, 0)),
            pl.BlockSpec((BR, 1), lambda i: (i, 0)),
            pl.BlockSpec((1, 256), lambda i: (0, 0)),
            pl.BlockSpec((256, dout), lambda i: (0, 0)),
        ],
        out_specs=out_spec,
        out_shape=out_shape,
    )(a, dis, b, W)


def _k4_body(a_ref, dis_ref, b_ref, o_ref):
    a = a_ref[0, :, :] + a_ref[1, :, :]
    o_ref[...] = jax.nn.relu(a * dis_ref[...] + b_ref[...])


def _k4(a, dis, b3):
    grid = NPAD // BR
    return pl.pallas_call(
        _k4_body,
        grid=(grid,),
        in_specs=[
            pl.BlockSpec((NC, BR, 128), lambda i: (0, i, 0)),
            pl.BlockSpec((BR, 1), lambda i: (i, 0)),
            pl.BlockSpec((1, 128), lambda i: (0, 0)),
        ],
        out_specs=pl.BlockSpec((BR, 128), lambda i: (i, 0)),
        out_shape=jax.ShapeDtypeStruct((NPAD, 128), jnp.float32),
    )(a, dis, b3)


def _pad_edges(src, dst, total):
    npad = total - src.shape[0]
    pad_src = (jnp.arange(npad, dtype=jnp.int32) * 181) % N
    pad_dst = N + (jnp.arange(npad, dtype=jnp.int32) % (NPAD - N))
    return (jnp.concatenate([src, pad_src]), jnp.concatenate([dst, pad_dst]))


def kernel(x, batch_edge_index, W1, b1, W2, b2, W3, b3):
    n_batch, n_points, _ = x.shape
    h0 = jnp.pad(x.reshape(-1, x.shape[-1]), ((0, NPAD - N), (0, 0)))

    loop = jnp.arange(N, dtype=jnp.int32)
    src = jnp.concatenate([batch_edge_index[0], loop])
    dst = jnp.concatenate([batch_edge_index[1], loop])

    srcc, dstc = _pad_edges(src, dst, E_COL)
    src_col = srcc.reshape(NS, T, B)
    src_off = jnp.stack([src_col, src_col + NPAD])
    dst_col = dstc.reshape(NS, T, B)

    srce, dste = _pad_edges(src, dst, E_EDG)
    src_edge = srce.reshape(NC * NS, TE, B)
    dst_edge = dste.reshape(NC * NS, TE, B)

    zeros128 = jnp.zeros((RPT, 128), jnp.float32)
    ones_tab = jnp.ones((NC * NPAD, 128), jnp.float32)

    deg_full = _scatter_col(ones_tab, src_off, dst_col, zeros128)
    dis, g1 = _k1(h0, deg_full, W1)
    a1 = _scatter_col(g1.reshape(NC * NPAD, 128), src_off, dst_col, zeros128)
    g2 = _mid(a1, dis, b1.reshape(1, 256), W2, split_out=True)
    a2 = _scatter_col(g2.reshape(NC * NPAD, 128), src_off, dst_col, zeros128)
    g3 = _mid(a2, dis, b2.reshape(1, 256), W3, split_out=False)
    a3 = _scatter_edge(g3, src_edge, dst_edge, zeros128)
    out = _k4(a3, dis, b3.reshape(1, 128))
    return out[:N].reshape(n_batch, n_points, -1)


# speedup vs baseline: 17.8093x; 1.1957x over previous
import functools

import jax
import jax.numpy as jnp
from jax import lax
from jax.experimental import pallas as pl
from jax.experimental.pallas import tpu as pltpu
from jax.experimental.pallas import tpu_sc as plsc

N = 10000
NPAD = 10240
NC, NS = 2, 16
B = 128
T = 168
CT = 56
TE = 96
CTE = 48
E_COL = NS * T * B
E_EDG = NC * NS * TE * B
RPT = NPAD // NS
BR = 1024

_MESH = plsc.VectorSubcoreMesh(core_axis_name="c", subcore_axis_name="s")


def _pipeline_chunk(g_hbm, acc, sidx, didx, buf0, buf1, sem0, sem1, n):
    pltpu.async_copy(g_hbm.at[sidx.at[0]], buf0, sem0)
    pltpu.async_copy(g_hbm.at[sidx.at[1]], buf1, sem1)

    @pl.loop(0, n, step=2)
    def _(j):
        pltpu.make_async_copy(g_hbm.at[sidx.at[j]], buf0, sem0).wait()
        pltpu.sync_copy(buf0, acc.at[didx.at[j]], add=True)

        @pl.when(j + 2 < n)
        def _():
            pltpu.async_copy(g_hbm.at[sidx.at[j + 2]], buf0, sem0)

        pltpu.make_async_copy(g_hbm.at[sidx.at[j + 1]], buf1, sem1).wait()
        pltpu.sync_copy(buf1, acc.at[didx.at[j + 1]], add=True)

        @pl.when(j + 3 < n)
        def _():
            pltpu.async_copy(g_hbm.at[sidx.at[j + 3]], buf1, sem1)


@functools.partial(
    pl.kernel,
    out_type=jax.ShapeDtypeStruct((NC, NPAD, 128), jnp.float32),
    mesh=_MESH,
    scratch_types=[
        pltpu.VMEM((CT, B), jnp.int32),
        pltpu.VMEM((CT, B), jnp.int32),
        pltpu.VMEM((B, 128), jnp.float32),
        pltpu.VMEM((B, 128), jnp.float32),
        pltpu.SemaphoreType.DMA,
        pltpu.SemaphoreType.DMA,
        pltpu.VMEM_SHARED((NPAD, 128), jnp.float32),
    ],
)
def _scatter_col(g_hbm, src_hbm, dst_hbm, zeros_hbm, out,
                 sidx, didx, buf0, buf1, sem0, sem1, acc):
    c = lax.axis_index("c")
    s = lax.axis_index("s")
    pltpu.sync_copy(zeros_hbm, acc.at[pl.ds(s * RPT, RPT)])
    plsc.subcore_barrier()

    for k in range(T // CT):
        pltpu.sync_copy(src_hbm.at[c, s, pl.ds(k * CT, CT)], sidx)
        pltpu.sync_copy(dst_hbm.at[s, pl.ds(k * CT, CT)], didx)
        _pipeline_chunk(g_hbm, acc, sidx, didx, buf0, buf1, sem0, sem1, CT)

    plsc.subcore_barrier()
    pltpu.sync_copy(acc.at[pl.ds(s * RPT, RPT)],
                    out.at[c, pl.ds(s * RPT, RPT), :])


@functools.partial(
    pl.kernel,
    out_type=jax.ShapeDtypeStruct((NC, NPAD, 128), jnp.float32),
    mesh=_MESH,
    scratch_types=[
        pltpu.VMEM((CTE, B), jnp.int32),
        pltpu.VMEM((CTE, B), jnp.int32),
        pltpu.VMEM((B, 128), jnp.float32),
        pltpu.VMEM((B, 128), jnp.float32),
        pltpu.SemaphoreType.DMA,
        pltpu.SemaphoreType.DMA,
        pltpu.VMEM_SHARED((NPAD, 128), jnp.float32),
    ],
)
def _scatter_edge(g_hbm, src_hbm, dst_hbm, zeros_hbm, out,
                  sidx, didx, buf0, buf1, sem0, sem1, acc):
    c = lax.axis_index("c")
    s = lax.axis_index("s")
    wid = c * NS + s
    pltpu.sync_copy(zeros_hbm, acc.at[pl.ds(s * RPT, RPT)])
    plsc.subcore_barrier()

    for k in range(TE // CTE):
        pltpu.sync_copy(src_hbm.at[wid, pl.ds(k * CTE, CTE)], sidx)
        pltpu.sync_copy(dst_hbm.at[wid, pl.ds(k * CTE, CTE)], didx)
        _pipeline_chunk(g_hbm, acc, sidx, didx, buf0, buf1, sem0, sem1, CTE)

    plsc.subcore_barrier()
    pltpu.sync_copy(acc.at[pl.ds(s * RPT, RPT)],
                    out.at[c, pl.ds(s * RPT, RPT), :])


@functools.partial(
    pl.kernel,
    out_type=jax.ShapeDtypeStruct((NC, NPAD, 128), jnp.float32),
    mesh=_MESH,
    scratch_types=[
        pltpu.VMEM((TE, B), jnp.int32),
        pltpu.VMEM((B, 128), jnp.float32),
        pltpu.VMEM_SHARED((NPAD, 128), jnp.float32),
    ],
)
def _deg_scatter(dst_hbm, zeros_hbm, ones_hbm, out, didx, ones_v, acc):
    c = lax.axis_index("c")
    s = lax.axis_index("s")
    wid = c * NS + s
    pltpu.sync_copy(zeros_hbm, acc.at[pl.ds(s * RPT, RPT)])
    pltpu.sync_copy(ones_hbm, ones_v)
    pltpu.sync_copy(dst_hbm.at[wid], didx)
    plsc.subcore_barrier()

    @pl.loop(0, TE)
    def _(j):
        pltpu.sync_copy(ones_v, acc.at[didx.at[j]], add=True)

    plsc.subcore_barrier()
    pltpu.sync_copy(acc.at[pl.ds(s * RPT, RPT)],
                    out.at[c, pl.ds(s * RPT, RPT), :])


def _k1_body(h_ref, deg_ref, w_ref, dis_ref, g_ref):
    deg = deg_ref[0, :, 0:1] + deg_ref[1, :, 0:1]
    dis = jnp.where(deg > 0.0, lax.rsqrt(deg), 0.0)
    dis_ref[...] = dis
    hw = jnp.dot(h_ref[...], w_ref[...],
                 preferred_element_type=jnp.float32,
                 precision=lax.Precision.HIGHEST) * dis
    g_ref[0, :, :] = hw[:, :128]
    g_ref[1, :, :] = hw[:, 128:]


def _k1(h, deg_full, W1):
    grid = NPAD // BR
    return pl.pallas_call(
        _k1_body,
        grid=(grid,),
        in_specs=[
            pl.BlockSpec((BR, 128), lambda i: (i, 0)),
            pl.BlockSpec((NC, BR, 128), lambda i: (0, i, 0)),
            pl.BlockSpec((128, 256), lambda i: (0, 0)),
        ],
        out_specs=[
            pl.BlockSpec((BR, 1), lambda i: (i, 0)),
            pl.BlockSpec((NC, BR, 128), lambda i: (0, i, 0)),
        ],
        out_shape=[
            jax.ShapeDtypeStruct((NPAD, 1), jnp.float32),
            jax.ShapeDtypeStruct((NC, NPAD, 128), jnp.float32),
        ],
    )(h, deg_full, W1)


def _mk_mid_body(split_out):
    def body(a_ref, dis_ref, b_ref, w_ref, g_ref):
        dis = dis_ref[...]
        s0 = jax.nn.relu(a_ref[0, :, :] * dis + b_ref[:, :128])
        s1 = jax.nn.relu(a_ref[1, :, :] * dis + b_ref[:, 128:])
        hw = (jnp.dot(s0, w_ref[:128, :],
                      preferred_element_type=jnp.float32,
                      precision=lax.Precision.HIGHEST)
              + jnp.dot(s1, w_ref[128:, :],
                        preferred_element_type=jnp.float32,
                        precision=lax.Precision.HIGHEST)) * dis
        if split_out:
            g_ref[0, :, :] = hw[:, :128]
            g_ref[1, :, :] = hw[:, 128:]
        else:
            g_ref[...] = hw
    return body


def _mid(a, dis, b, W, split_out):
    grid = NPAD // BR
    dout = W.shape[1]
    if split_out:
        out_spec = pl.BlockSpec((NC, BR, 128), lambda i: (0, i, 0))
        out_shape = jax.ShapeDtypeStruct((NC, NPAD, 128), jnp.float32)
    else:
        out_spec = pl.BlockSpec((BR, dout), lambda i: (i, 0))
        out_shape = jax.ShapeDtypeStruct((NPAD, dout), jnp.float32)
    return pl.pallas_call(
        _mk_mid_body(split_out),
        grid=(grid,),
        in_specs=[
            pl.BlockSpec((NC, BR, 128), lambda i: (0, i, 0)),
            pl.BlockSpec((BR, 1), lambda i: (i, 0)),
            pl.BlockSpec((1, 256), lambda i: (0, 0)),
            pl.BlockSpec((256, dout), lambda i: (0, 0)),
        ],
        out_specs=out_spec,
        out_shape=out_shape,
    )(a, dis, b, W)


def _k4_body(a_ref, dis_ref, b_ref, o_ref):
    a = a_ref[0, :, :] + a_ref[1, :, :]
    o_ref[...] = jax.nn.relu(a * dis_ref[...] + b_ref[...])


def _k4(a, dis, b3):
    grid = NPAD // BR
    return pl.pallas_call(
        _k4_body,
        grid=(grid,),
        in_specs=[
            pl.BlockSpec((NC, BR, 128), lambda i: (0, i, 0)),
            pl.BlockSpec((BR, 1), lambda i: (i, 0)),
            pl.BlockSpec((1, 128), lambda i: (0, 0)),
        ],
        out_specs=pl.BlockSpec((BR, 128), lambda i: (i, 0)),
        out_shape=jax.ShapeDtypeStruct((NPAD, 128), jnp.float32),
    )(a, dis, b3)


def _pad_edges(src, dst, total):
    npad = total - src.shape[0]
    pad_src = (jnp.arange(npad, dtype=jnp.int32) * 181) % N
    pad_dst = N + (jnp.arange(npad, dtype=jnp.int32) % (NPAD - N))
    return (jnp.concatenate([src, pad_src]), jnp.concatenate([dst, pad_dst]))


def kernel(x, batch_edge_index, W1, b1, W2, b2, W3, b3):
    n_batch, n_points, _ = x.shape
    h0 = jnp.pad(x.reshape(-1, x.shape[-1]), ((0, NPAD - N), (0, 0)))

    loop = jnp.arange(N, dtype=jnp.int32)
    src = jnp.concatenate([batch_edge_index[0], loop])
    dst = jnp.concatenate([batch_edge_index[1], loop])

    srcc, dstc = _pad_edges(src, dst, E_COL)
    src_col = srcc.reshape(NS, T, B)
    src_off = jnp.stack([src_col, src_col + NPAD])
    dst_col = dstc.reshape(NS, T, B)

    srce, dste = _pad_edges(src, dst, E_EDG)
    src_edge = srce.reshape(NC * NS, TE, B)
    dst_edge = dste.reshape(NC * NS, TE, B)

    zeros128 = jnp.zeros((RPT, 128), jnp.float32)
    ones_b = jnp.ones((B, 128), jnp.float32)

    deg_parts = _deg_scatter(dst_edge, zeros128, ones_b)
    dis, g1 = _k1(h0, deg_parts, W1)
    a1 = _scatter_col(g1.reshape(NC * NPAD, 128), src_off, dst_col, zeros128)
    g2 = _mid(a1, dis, b1.reshape(1, 256), W2, split_out=True)
    a2 = _scatter_col(g2.reshape(NC * NPAD, 128), src_off, dst_col, zeros128)
    g3 = _mid(a2, dis, b2.reshape(1, 256), W3, split_out=False)
    a3 = _scatter_edge(g3, src_edge, dst_edge, zeros128)
    out = _k4(a3, dis, b3.reshape(1, 128))
    return out[:N].reshape(n_batch, n_points, -1)


# speedup vs baseline: 18.1088x; 1.0168x over previous
import functools

import jax
import jax.numpy as jnp
from jax import lax
from jax.experimental import pallas as pl
from jax.experimental.pallas import tpu as pltpu
from jax.experimental.pallas import tpu_sc as plsc

N = 10000
NPAD = 10240
NC, NS = 2, 16
B = 128
T = 168
CT = 56
TER = 112
CTER = 56
BE = 96
E_COL = NS * T * B
E_EDG = NC * NS * TER * BE
RPT = NPAD // NS
BR = 1024

_MESH = plsc.VectorSubcoreMesh(core_axis_name="c", subcore_axis_name="s")


def _pipeline_chunk(g_hbm, acc, sidx, didx, buf0, buf1, sem0, sem1, n):
    pltpu.async_copy(g_hbm.at[sidx.at[0]], buf0, sem0)
    pltpu.async_copy(g_hbm.at[sidx.at[1]], buf1, sem1)

    @pl.loop(0, n, step=2)
    def _(j):
        pltpu.make_async_copy(g_hbm.at[sidx.at[j]], buf0, sem0).wait()
        pltpu.sync_copy(buf0, acc.at[didx.at[j]], add=True)

        @pl.when(j + 2 < n)
        def _():
            pltpu.async_copy(g_hbm.at[sidx.at[j + 2]], buf0, sem0)

        pltpu.make_async_copy(g_hbm.at[sidx.at[j + 1]], buf1, sem1).wait()
        pltpu.sync_copy(buf1, acc.at[didx.at[j + 1]], add=True)

        @pl.when(j + 3 < n)
        def _():
            pltpu.async_copy(g_hbm.at[sidx.at[j + 3]], buf1, sem1)


@functools.partial(
    pl.kernel,
    out_type=jax.ShapeDtypeStruct((NC, NPAD, 128), jnp.float32),
    mesh=_MESH,
    scratch_types=[
        pltpu.VMEM((CT, B), jnp.int32),
        pltpu.VMEM((CT, B), jnp.int32),
        pltpu.VMEM((B, 128), jnp.float32),
        pltpu.VMEM((B, 128), jnp.float32),
        pltpu.SemaphoreType.DMA,
        pltpu.SemaphoreType.DMA,
        pltpu.VMEM_SHARED((NPAD, 128), jnp.float32),
    ],
)
def _scatter_col(g_hbm, src_hbm, dst_hbm, zeros_hbm, out,
                 sidx, didx, buf0, buf1, sem0, sem1, acc):
    c = lax.axis_index("c")
    s = lax.axis_index("s")
    pltpu.sync_copy(zeros_hbm, acc.at[pl.ds(s * RPT, RPT)])
    plsc.subcore_barrier()

    for k in range(T // CT):
        pltpu.sync_copy(src_hbm.at[c, s, pl.ds(k * CT, CT)], sidx)
        pltpu.sync_copy(dst_hbm.at[s, pl.ds(k * CT, CT)], didx)
        _pipeline_chunk(g_hbm, acc, sidx, didx, buf0, buf1, sem0, sem1, CT)

    plsc.subcore_barrier()
    pltpu.sync_copy(acc.at[pl.ds(s * RPT, RPT)],
                    out.at[c, pl.ds(s * RPT, RPT), :])


@functools.partial(
    pl.kernel,
    out_type=jax.ShapeDtypeStruct((NC, NPAD, 128), jnp.float32),
    mesh=_MESH,
    scratch_types=[
        pltpu.VMEM((CTER, BE), jnp.int32),
        pltpu.VMEM((CTER, BE), jnp.int32),
        pltpu.VMEM((BE, 128), jnp.float32),
        pltpu.VMEM((BE, 128), jnp.float32),
        pltpu.SemaphoreType.DMA,
        pltpu.SemaphoreType.DMA,
        pltpu.VMEM_SHARED((NPAD, 128), jnp.float32),
    ],
)
def _scatter_edge(g_hbm, src_hbm, dst_hbm, zeros_hbm, out,
                  sidx, didx, buf0, buf1, sem0, sem1, acc):
    c = lax.axis_index("c")
    s = lax.axis_index("s")
    wid = c * NS + s
    pltpu.sync_copy(zeros_hbm, acc.at[pl.ds(s * RPT, RPT)])
    plsc.subcore_barrier()

    for k in range(TER // CTER):
        pltpu.sync_copy(src_hbm.at[wid, pl.ds(k * CTER, CTER)], sidx)
        pltpu.sync_copy(dst_hbm.at[wid, pl.ds(k * CTER, CTER)], didx)
        _pipeline_chunk(g_hbm, acc, sidx, didx, buf0, buf1, sem0, sem1, CTER)

    plsc.subcore_barrier()
    pltpu.sync_copy(acc.at[pl.ds(s * RPT, RPT)],
                    out.at[c, pl.ds(s * RPT, RPT), :])


@functools.partial(
    pl.kernel,
    out_type=jax.ShapeDtypeStruct((NC, NPAD, 128), jnp.float32),
    mesh=_MESH,
    scratch_types=[
        pltpu.VMEM((TER, BE), jnp.int32),
        pltpu.VMEM((BE, 128), jnp.float32),
        pltpu.VMEM_SHARED((NPAD, 128), jnp.float32),
    ],
)
def _deg_scatter(dst_hbm, zeros_hbm, ones_hbm, out, didx, ones_v, acc):
    c = lax.axis_index("c")
    s = lax.axis_index("s")
    wid = c * NS + s
    pltpu.sync_copy(zeros_hbm, acc.at[pl.ds(s * RPT, RPT)])
    pltpu.sync_copy(ones_hbm, ones_v)
    pltpu.sync_copy(dst_hbm.at[wid], didx)
    plsc.subcore_barrier()

    @pl.loop(0, TER)
    def _(j):
        pltpu.sync_copy(ones_v, acc.at[didx.at[j]], add=True)

    plsc.subcore_barrier()
    pltpu.sync_copy(acc.at[pl.ds(s * RPT, RPT)],
                    out.at[c, pl.ds(s * RPT, RPT), :])


def _k1_body(h_ref, deg_ref, w_ref, dis_ref, g_ref):
    deg = deg_ref[0, :, 0:1] + deg_ref[1, :, 0:1]
    dis = jnp.where(deg > 0.0, lax.rsqrt(deg), 0.0)
    dis_ref[...] = dis
    hw = jnp.dot(h_ref[...], w_ref[...],
                 preferred_element_type=jnp.float32,
                 precision=lax.Precision.HIGHEST) * dis
    g_ref[0, :, :] = hw[:, :128]
    g_ref[1, :, :] = hw[:, 128:]


def _k1(h, deg_full, W1):
    grid = NPAD // BR
    return pl.pallas_call(
        _k1_body,
        grid=(grid,),
        in_specs=[
            pl.BlockSpec((BR, 128), lambda i: (i, 0)),
            pl.BlockSpec((NC, BR, 128), lambda i: (0, i, 0)),
            pl.BlockSpec((128, 256), lambda i: (0, 0)),
        ],
        out_specs=[
            pl.BlockSpec((BR, 1), lambda i: (i, 0)),
            pl.BlockSpec((NC, BR, 128), lambda i: (0, i, 0)),
        ],
        out_shape=[
            jax.ShapeDtypeStruct((NPAD, 1), jnp.float32),
            jax.ShapeDtypeStruct((NC, NPAD, 128), jnp.float32),
        ],
    )(h, deg_full, W1)


def _mk_mid_body(split_out):
    def body(a_ref, dis_ref, b_ref, w_ref, g_ref):
        dis = dis_ref[...]
        s0 = jax.nn.relu(a_ref[0, :, :] * dis + b_ref[:, :128])
        s1 = jax.nn.relu(a_ref[1, :, :] * dis + b_ref[:, 128:])
        hw = (jnp.dot(s0, w_ref[:128, :],
                      preferred_element_type=jnp.float32,
                      precision=lax.Precision.HIGHEST)
              + jnp.dot(s1, w_ref[128:, :],
                        preferred_element_type=jnp.float32,
                        precision=lax.Precision.HIGHEST)) * dis
        if split_out:
            g_ref[0, :, :] = hw[:, :128]
            g_ref[1, :, :] = hw[:, 128:]
        else:
            g_ref[...] = hw
    return body


def _mid(a, dis, b, W, split_out):
    grid = NPAD // BR
    dout = W.shape[1]
    if split_out:
        out_spec = pl.BlockSpec((NC, BR, 128), lambda i: (0, i, 0))
        out_shape = jax.ShapeDtypeStruct((NC, NPAD, 128), jnp.float32)
    else:
        out_spec = pl.BlockSpec((BR, dout), lambda i: (i, 0))
        out_shape = jax.ShapeDtypeStruct((NPAD, dout), jnp.float32)
    return pl.pallas_call(
        _mk_mid_body(split_out),
        grid=(grid,),
        in_specs=[
            pl.BlockSpec((NC, BR, 128), lambda i: (0, i, 0)),
            pl.BlockSpec((BR, 1), lambda i: (i, 0)),
            pl.BlockSpec((1, 256), lambda i: (0, 0)),
            pl.BlockSpec((256, dout), lambda i: (0, 0)),
        ],
        out_specs=out_spec,
        out_shape=out_shape,
    )(a, dis, b, W)


def _k4_body(a_ref, dis_ref, b_ref, o_ref):
    a = a_ref[0, :, :] + a_ref[1, :, :]
    o_ref[...] = jax.nn.relu(a * dis_ref[...] + b_ref[...])


def _k4(a, dis, b3):
    grid = NPAD // BR
    return pl.pallas_call(
        _k4_body,
        grid=(grid,),
        in_specs=[
            pl.BlockSpec((NC, BR, 128), lambda i: (0, i, 0)),
            pl.BlockSpec((BR, 1), lambda i: (i, 0)),
            pl.BlockSpec((1, 128), lambda i: (0, 0)),
        ],
        out_specs=pl.BlockSpec((BR, 128), lambda i: (i, 0)),
        out_shape=jax.ShapeDtypeStruct((NPAD, 128), jnp.float32),
    )(a, dis, b3)


def _pad_edges(src, dst, total):
    npad = total - src.shape[0]
    pad_src = (jnp.arange(npad, dtype=jnp.int32) * 181) % N
    pad_dst = N + (jnp.arange(npad, dtype=jnp.int32) % (NPAD - N))
    return (jnp.concatenate([src, pad_src]), jnp.concatenate([dst, pad_dst]))


def kernel(x, batch_edge_index, W1, b1, W2, b2, W3, b3):
    n_batch, n_points, _ = x.shape
    h0 = jnp.pad(x.reshape(-1, x.shape[-1]), ((0, NPAD - N), (0, 0)))

    loop = jnp.arange(N, dtype=jnp.int32)
    src = jnp.concatenate([batch_edge_index[0], loop])
    dst = jnp.concatenate([batch_edge_index[1], loop])

    srcc, dstc = _pad_edges(src, dst, E_COL)
    src_col = srcc.reshape(NS, T, B)
    src_off = jnp.stack([src_col, src_col + NPAD])
    dst_col = dstc.reshape(NS, T, B)

    src_edge = srcc.reshape(NC * NS, TER, BE)
    dst_edge = dstc.reshape(NC * NS, TER, BE)

    zeros128 = jnp.zeros((RPT, 128), jnp.float32)
    ones_b = jnp.ones((BE, 128), jnp.float32)

    deg_parts = _deg_scatter(dst_edge, zeros128, ones_b)
    dis, g1 = _k1(h0, deg_parts, W1)
    a1 = _scatter_col(g1.reshape(NC * NPAD, 128), src_off, dst_col, zeros128)
    g2 = _mid(a1, dis, b1.reshape(1, 256), W2, split_out=True)
    a2 = _scatter_col(g2.reshape(NC * NPAD, 128), src_off, dst_col, zeros128)
    g3 = _mid(a2, dis, b2.reshape(1, 256), W3, split_out=False)
    a3 = _scatter_edge(g3, src_edge, dst_edge, zeros128)
    out = _k4(a3, dis, b3.reshape(1, 128))
    return out[:N].reshape(n_batch, n_points, -1)


# speedup vs baseline: 20.2222x; 1.1167x over previous
import functools

import jax
import jax.numpy as jnp
from jax import lax
from jax.experimental import pallas as pl
from jax.experimental.pallas import tpu as pltpu
from jax.experimental.pallas import tpu_sc as plsc

N = 10000
NPAD = 10240
NC, NS = 2, 16
B = 128
T = 168
CT = 56
TER = 112
CTER = 56
BE = 96
E_COL = NS * T * B
E_EDG = NC * NS * TER * BE
RPT = NPAD // NS
BR = 1024

_MESH = plsc.VectorSubcoreMesh(core_axis_name="c", subcore_axis_name="s")


def _pipeline_chunk(g_hbm, acc, sidx, didx, buf0, buf1, sem0, sem1, n):
    pltpu.async_copy(g_hbm.at[sidx.at[0]], buf0, sem0)
    pltpu.async_copy(g_hbm.at[sidx.at[1]], buf1, sem1)

    @pl.loop(0, n, step=2)
    def _(j):
        pltpu.make_async_copy(g_hbm.at[sidx.at[j]], buf0, sem0).wait()
        pltpu.sync_copy(buf0, acc.at[didx.at[j]], add=True)

        @pl.when(j + 2 < n)
        def _():
            pltpu.async_copy(g_hbm.at[sidx.at[j + 2]], buf0, sem0)

        pltpu.make_async_copy(g_hbm.at[sidx.at[j + 1]], buf1, sem1).wait()
        pltpu.sync_copy(buf1, acc.at[didx.at[j + 1]], add=True)

        @pl.when(j + 3 < n)
        def _():
            pltpu.async_copy(g_hbm.at[sidx.at[j + 3]], buf1, sem1)


@functools.partial(
    pl.kernel,
    out_type=jax.ShapeDtypeStruct((NC, NPAD, 128), jnp.float32),
    mesh=_MESH,
    scratch_types=[
        pltpu.VMEM((CT, B), jnp.int32),
        pltpu.VMEM((CT, B), jnp.int32),
        pltpu.VMEM((B, 128), jnp.float32),
        pltpu.VMEM((B, 128), jnp.float32),
        pltpu.SemaphoreType.DMA,
        pltpu.SemaphoreType.DMA,
        pltpu.VMEM_SHARED((NPAD, 128), jnp.float32),
    ],
)
def _scatter_col(g_hbm, src_hbm, dst_hbm, zeros_hbm, out,
                 sidx, didx, buf0, buf1, sem0, sem1, acc):
    c = lax.axis_index("c")
    s = lax.axis_index("s")
    pltpu.sync_copy(zeros_hbm, acc.at[pl.ds(s * RPT, RPT)])
    plsc.subcore_barrier()

    for k in range(T // CT):
        pltpu.sync_copy(src_hbm.at[c, s, pl.ds(k * CT, CT)], sidx)
        pltpu.sync_copy(dst_hbm.at[s, pl.ds(k * CT, CT)], didx)
        _pipeline_chunk(g_hbm, acc, sidx, didx, buf0, buf1, sem0, sem1, CT)

    plsc.subcore_barrier()
    pltpu.sync_copy(acc.at[pl.ds(s * RPT, RPT)],
                    out.at[c, pl.ds(s * RPT, RPT), :])


@functools.partial(
    pl.kernel,
    out_type=jax.ShapeDtypeStruct((NC, NPAD, 128), jnp.float32),
    mesh=_MESH,
    scratch_types=[
        pltpu.VMEM((CTER, BE), jnp.int32),
        pltpu.VMEM((CTER, BE), jnp.int32),
        pltpu.VMEM((BE, 128), jnp.float32),
        pltpu.VMEM((BE, 128), jnp.float32),
        pltpu.SemaphoreType.DMA,
        pltpu.SemaphoreType.DMA,
        pltpu.VMEM_SHARED((NPAD, 128), jnp.float32),
    ],
)
def _scatter_edge(g_hbm, src_hbm, dst_hbm, zeros_hbm, out,
                  sidx, didx, buf0, buf1, sem0, sem1, acc):
    c = lax.axis_index("c")
    s = lax.axis_index("s")
    wid = c * NS + s
    pltpu.sync_copy(zeros_hbm, acc.at[pl.ds(s * RPT, RPT)])
    plsc.subcore_barrier()

    for k in range(TER // CTER):
        pltpu.sync_copy(src_hbm.at[wid, pl.ds(k * CTER, CTER)], sidx)
        pltpu.sync_copy(dst_hbm.at[wid, pl.ds(k * CTER, CTER)], didx)
        _pipeline_chunk(g_hbm, acc, sidx, didx, buf0, buf1, sem0, sem1, CTER)

    plsc.subcore_barrier()
    pltpu.sync_copy(acc.at[pl.ds(s * RPT, RPT)],
                    out.at[c, pl.ds(s * RPT, RPT), :])


@functools.partial(
    pl.kernel,
    out_type=jax.ShapeDtypeStruct((NC, NPAD, 128), jnp.float32),
    mesh=_MESH,
    scratch_types=[
        pltpu.VMEM((TER, BE), jnp.int32),
        pltpu.VMEM((BE, 128), jnp.float32),
        pltpu.VMEM_SHARED((NPAD, 128), jnp.float32),
    ],
)
def _deg_scatter(dst_hbm, zeros_hbm, ones_hbm, out, didx, ones_v, acc):
    c = lax.axis_index("c")
    s = lax.axis_index("s")
    wid = c * NS + s
    pltpu.sync_copy(zeros_hbm, acc.at[pl.ds(s * RPT, RPT)])
    pltpu.sync_copy(ones_hbm, ones_v)
    pltpu.sync_copy(dst_hbm.at[wid], didx)
    plsc.subcore_barrier()

    @pl.loop(0, TER)
    def _(j):
        pltpu.sync_copy(ones_v, acc.at[didx.at[j]], add=True)

    plsc.subcore_barrier()
    pltpu.sync_copy(acc.at[pl.ds(s * RPT, RPT)],
                    out.at[c, pl.ds(s * RPT, RPT), :])


def _k1_body(x_ref, deg_ref, dis_ref, z_ref):
    deg = deg_ref[0, :, 0:1] + deg_ref[1, :, 0:1]
    dis = jnp.where(deg > 0.0, lax.rsqrt(deg), 0.0)
    dis_ref[...] = dis
    z_ref[...] = x_ref[...] * dis


def _k1(h, deg_parts):
    grid = NPAD // BR
    return pl.pallas_call(
        _k1_body,
        grid=(grid,),
        in_specs=[
            pl.BlockSpec((BR, 128), lambda i: (i, 0)),
            pl.BlockSpec((NC, BR, 128), lambda i: (0, i, 0)),
        ],
        out_specs=[
            pl.BlockSpec((BR, 1), lambda i: (i, 0)),
            pl.BlockSpec((BR, 128), lambda i: (i, 0)),
        ],
        out_shape=[
            jax.ShapeDtypeStruct((NPAD, 1), jnp.float32),
            jax.ShapeDtypeStruct((NPAD, 128), jnp.float32),
        ],
    )(h, deg_parts)


def _k2_body(p_ref, dis_ref, b_ref, w1_ref, w2_ref, g_ref):
    dis = dis_ref[...]
    agg = p_ref[0, :, :] + p_ref[1, :, :]
    s1 = jax.nn.relu(jnp.dot(agg, w1_ref[...],
                             preferred_element_type=jnp.float32,
                             precision=lax.Precision.HIGHEST) * dis + b_ref[...])
    hw = (jnp.dot(s1[:, :128], w2_ref[:128, :],
                  preferred_element_type=jnp.float32,
                  precision=lax.Precision.HIGHEST)
          + jnp.dot(s1[:, 128:], w2_ref[128:, :],
                    preferred_element_type=jnp.float32,
                    precision=lax.Precision.HIGHEST)) * dis
    g_ref[0, :, :] = hw[:, :128]
    g_ref[1, :, :] = hw[:, 128:]


def _k2(p, dis, b1, W1, W2):
    grid = NPAD // BR
    return pl.pallas_call(
        _k2_body,
        grid=(grid,),
        in_specs=[
            pl.BlockSpec((NC, BR, 128), lambda i: (0, i, 0)),
            pl.BlockSpec((BR, 1), lambda i: (i, 0)),
            pl.BlockSpec((1, 256), lambda i: (0, 0)),
            pl.BlockSpec((128, 256), lambda i: (0, 0)),
            pl.BlockSpec((256, 256), lambda i: (0, 0)),
        ],
        out_specs=pl.BlockSpec((NC, BR, 128), lambda i: (0, i, 0)),
        out_shape=jax.ShapeDtypeStruct((NC, NPAD, 128), jnp.float32),
    )(p, dis, b1, W1, W2)


def _mk_mid_body(split_out):
    def body(a_ref, dis_ref, b_ref, w_ref, g_ref):
        dis = dis_ref[...]
        s0 = jax.nn.relu(a_ref[0, :, :] * dis + b_ref[:, :128])
        s1 = jax.nn.relu(a_ref[1, :, :] * dis + b_ref[:, 128:])
        hw = (jnp.dot(s0, w_ref[:128, :],
                      preferred_element_type=jnp.float32,
                      precision=lax.Precision.HIGHEST)
              + jnp.dot(s1, w_ref[128:, :],
                        preferred_element_type=jnp.float32,
                        precision=lax.Precision.HIGHEST)) * dis
        if split_out:
            g_ref[0, :, :] = hw[:, :128]
            g_ref[1, :, :] = hw[:, 128:]
        else:
            g_ref[...] = hw
    return body


def _mid(a, dis, b, W, split_out):
    grid = NPAD // BR
    dout = W.shape[1]
    if split_out:
        out_spec = pl.BlockSpec((NC, BR, 128), lambda i: (0, i, 0))
        out_shape = jax.ShapeDtypeStruct((NC, NPAD, 128), jnp.float32)
    else:
        out_spec = pl.BlockSpec((BR, dout), lambda i: (i, 0))
        out_shape = jax.ShapeDtypeStruct((NPAD, dout), jnp.float32)
    return pl.pallas_call(
        _mk_mid_body(split_out),
        grid=(grid,),
        in_specs=[
            pl.BlockSpec((NC, BR, 128), lambda i: (0, i, 0)),
            pl.BlockSpec((BR, 1), lambda i: (i, 0)),
            pl.BlockSpec((1, 256), lambda i: (0, 0)),
            pl.BlockSpec((256, dout), lambda i: (0, 0)),
        ],
        out_specs=out_spec,
        out_shape=out_shape,
    )(a, dis, b, W)


def _k4_body(a_ref, dis_ref, b_ref, o_ref):
    a = a_ref[0, :, :] + a_ref[1, :, :]
    o_ref[...] = jax.nn.relu(a * dis_ref[...] + b_ref[...])


def _k4(a, dis, b3):
    grid = NPAD // BR
    return pl.pallas_call(
        _k4_body,
        grid=(grid,),
        in_specs=[
            pl.BlockSpec((NC, BR, 128), lambda i: (0, i, 0)),
            pl.BlockSpec((BR, 1), lambda i: (i, 0)),
            pl.BlockSpec((1, 128), lambda i: (0, 0)),
        ],
        out_specs=pl.BlockSpec((BR, 128), lambda i: (i, 0)),
        out_shape=jax.ShapeDtypeStruct((NPAD, 128), jnp.float32),
    )(a, dis, b3)


def _pad_edges(src, dst, total):
    npad = total - src.shape[0]
    pad_src = (jnp.arange(npad, dtype=jnp.int32) * 181) % N
    pad_dst = N + (jnp.arange(npad, dtype=jnp.int32) % (NPAD - N))
    return (jnp.concatenate([src, pad_src]), jnp.concatenate([dst, pad_dst]))


def kernel(x, batch_edge_index, W1, b1, W2, b2, W3, b3):
    n_batch, n_points, _ = x.shape
    h0 = jnp.pad(x.reshape(-1, x.shape[-1]), ((0, NPAD - N), (0, 0)))

    loop = jnp.arange(N, dtype=jnp.int32)
    src = jnp.concatenate([batch_edge_index[0], loop])
    dst = jnp.concatenate([batch_edge_index[1], loop])

    srcc, dstc = _pad_edges(src, dst, E_COL)
    src_col = srcc.reshape(NS, T, B)
    src_off = jnp.stack([src_col, src_col + NPAD])
    dst_col = dstc.reshape(NS, T, B)

    src_edge = srcc.reshape(NC * NS, TER, BE)
    dst_edge = dstc.reshape(NC * NS, TER, BE)

    zeros128 = jnp.zeros((RPT, 128), jnp.float32)
    ones_b = jnp.ones((BE, 128), jnp.float32)

    deg_parts = _deg_scatter(dst_edge, zeros128, ones_b)
    dis, z = _k1(h0, deg_parts)
    p1 = _scatter_edge(z, src_edge, dst_edge, zeros128)
    g2 = _k2(p1, dis, b1.reshape(1, 256), W1, W2)
    a2 = _scatter_col(g2.reshape(NC * NPAD, 128), src_off, dst_col, zeros128)
    g3 = _mid(a2, dis, b2.reshape(1, 256), W3, split_out=False)
    a3 = _scatter_edge(g3, src_edge, dst_edge, zeros128)
    out = _k4(a3, dis, b3.reshape(1, 128))
    return out[:N].reshape(n_batch, n_points, -1)


# speedup vs baseline: 20.6692x; 1.0221x over previous
import functools

import jax
import jax.numpy as jnp
from jax import lax
from jax.experimental import pallas as pl
from jax.experimental.pallas import tpu as pltpu
from jax.experimental.pallas import tpu_sc as plsc

N = 10000
NPAD = 10240
NC, NS = 2, 16
B = 64
T = 336
CT = 56
TER = 112
CTER = 56
BE = 96
E_COL = NS * T * B
E_EDG = NC * NS * TER * BE
RPT = NPAD // NS
BR = 1024

_MESH = plsc.VectorSubcoreMesh(core_axis_name="c", subcore_axis_name="s")


def _pipeline_chunk(g_hbm, acc, sidx, didx, bufs, sems, n):
    nb = len(bufs)
    for i in range(nb):
        pltpu.async_copy(g_hbm.at[sidx.at[i]], bufs[i], sems[i])

    @pl.loop(0, n, step=nb)
    def _(j):
        for i in range(nb):
            pltpu.make_async_copy(g_hbm.at[sidx.at[j + i]],
                                  bufs[i], sems[i]).wait()
            pltpu.sync_copy(bufs[i], acc.at[didx.at[j + i]], add=True)

            @pl.when(j + nb + i < n)
            def _(i=i):
                pltpu.async_copy(g_hbm.at[sidx.at[j + nb + i]],
                                 bufs[i], sems[i])


@functools.partial(
    pl.kernel,
    out_type=jax.ShapeDtypeStruct((NC, NPAD, 128), jnp.float32),
    mesh=_MESH,
    scratch_types=[
        pltpu.VMEM((CT, B), jnp.int32),
        pltpu.VMEM((CT, B), jnp.int32),
        pltpu.VMEM((B, 128), jnp.float32),
        pltpu.VMEM((B, 128), jnp.float32),
        pltpu.VMEM((B, 128), jnp.float32),
        pltpu.VMEM((B, 128), jnp.float32),
        pltpu.SemaphoreType.DMA,
        pltpu.SemaphoreType.DMA,
        pltpu.SemaphoreType.DMA,
        pltpu.SemaphoreType.DMA,
        pltpu.VMEM_SHARED((NPAD, 128), jnp.float32),
    ],
)
def _scatter_col(g_hbm, src_hbm, dst_hbm, zeros_hbm, out,
                 sidx, didx, buf0, buf1, buf2, buf3,
                 sem0, sem1, sem2, sem3, acc):
    c = lax.axis_index("c")
    s = lax.axis_index("s")
    pltpu.sync_copy(zeros_hbm, acc.at[pl.ds(s * RPT, RPT)])
    plsc.subcore_barrier()

    for k in range(T // CT):
        pltpu.sync_copy(src_hbm.at[c, s, pl.ds(k * CT, CT)], sidx)
        pltpu.sync_copy(dst_hbm.at[s, pl.ds(k * CT, CT)], didx)
        _pipeline_chunk(g_hbm, acc, sidx, didx,
                        (buf0, buf1, buf2, buf3),
                        (sem0, sem1, sem2, sem3), CT)

    plsc.subcore_barrier()
    pltpu.sync_copy(acc.at[pl.ds(s * RPT, RPT)],
                    out.at[c, pl.ds(s * RPT, RPT), :])


@functools.partial(
    pl.kernel,
    out_type=jax.ShapeDtypeStruct((NC, NPAD, 128), jnp.float32),
    mesh=_MESH,
    scratch_types=[
        pltpu.VMEM((CTER, BE), jnp.int32),
        pltpu.VMEM((CTER, BE), jnp.int32),
        pltpu.VMEM((BE, 128), jnp.float32),
        pltpu.VMEM((BE, 128), jnp.float32),
        pltpu.SemaphoreType.DMA,
        pltpu.SemaphoreType.DMA,
        pltpu.VMEM_SHARED((NPAD, 128), jnp.float32),
    ],
)
def _scatter_edge(g_hbm, src_hbm, dst_hbm, zeros_hbm, out,
                  sidx, didx, buf0, buf1, sem0, sem1, acc):
    c = lax.axis_index("c")
    s = lax.axis_index("s")
    wid = c * NS + s
    pltpu.sync_copy(zeros_hbm, acc.at[pl.ds(s * RPT, RPT)])
    plsc.subcore_barrier()

    for k in range(TER // CTER):
        pltpu.sync_copy(src_hbm.at[wid, pl.ds(k * CTER, CTER)], sidx)
        pltpu.sync_copy(dst_hbm.at[wid, pl.ds(k * CTER, CTER)], didx)
        _pipeline_chunk(g_hbm, acc, sidx, didx,
                        (buf0, buf1), (sem0, sem1), CTER)

    plsc.subcore_barrier()
    pltpu.sync_copy(acc.at[pl.ds(s * RPT, RPT)],
                    out.at[c, pl.ds(s * RPT, RPT), :])


@functools.partial(
    pl.kernel,
    out_type=jax.ShapeDtypeStruct((NC, NPAD, 128), jnp.float32),
    mesh=_MESH,
    scratch_types=[
        pltpu.VMEM((TER, BE), jnp.int32),
        pltpu.VMEM((BE, 128), jnp.float32),
        pltpu.VMEM_SHARED((NPAD, 128), jnp.float32),
    ],
)
def _deg_scatter(dst_hbm, zeros_hbm, ones_hbm, out, didx, ones_v, acc):
    c = lax.axis_index("c")
    s = lax.axis_index("s")
    wid = c * NS + s
    pltpu.sync_copy(zeros_hbm, acc.at[pl.ds(s * RPT, RPT)])
    pltpu.sync_copy(ones_hbm, ones_v)
    pltpu.sync_copy(dst_hbm.at[wid], didx)
    plsc.subcore_barrier()

    @pl.loop(0, TER)
    def _(j):
        pltpu.sync_copy(ones_v, acc.at[didx.at[j]], add=True)

    plsc.subcore_barrier()
    pltpu.sync_copy(acc.at[pl.ds(s * RPT, RPT)],
                    out.at[c, pl.ds(s * RPT, RPT), :])


def _k1_body(x_ref, deg_ref, dis_ref, z_ref):
    deg = deg_ref[0, :, 0:1] + deg_ref[1, :, 0:1]
    dis = jnp.where(deg > 0.0, lax.rsqrt(deg), 0.0)
    dis_ref[...] = dis
    z_ref[...] = x_ref[...] * dis


def _k1(h, deg_parts):
    grid = NPAD // BR
    return pl.pallas_call(
        _k1_body,
        grid=(grid,),
        in_specs=[
            pl.BlockSpec((BR, 128), lambda i: (i, 0)),
            pl.BlockSpec((NC, BR, 128), lambda i: (0, i, 0)),
        ],
        out_specs=[
            pl.BlockSpec((BR, 1), lambda i: (i, 0)),
            pl.BlockSpec((BR, 128), lambda i: (i, 0)),
        ],
        out_shape=[
            jax.ShapeDtypeStruct((NPAD, 1), jnp.float32),
            jax.ShapeDtypeStruct((NPAD, 128), jnp.float32),
        ],
    )(h, deg_parts)


def _k2_body(p_ref, dis_ref, b_ref, w1_ref, w2_ref, g_ref):
    dis = dis_ref[...]
    agg = p_ref[0, :, :] + p_ref[1, :, :]
    s1 = jax.nn.relu(jnp.dot(agg, w1_ref[...],
                             preferred_element_type=jnp.float32,
                             precision=lax.Precision.HIGHEST) * dis + b_ref[...])
    hw = (jnp.dot(s1[:, :128], w2_ref[:128, :],
                  preferred_element_type=jnp.float32,
                  precision=lax.Precision.HIGHEST)
          + jnp.dot(s1[:, 128:], w2_ref[128:, :],
                    preferred_element_type=jnp.float32,
                    precision=lax.Precision.HIGHEST)) * dis
    g_ref[0, :, :] = hw[:, :128]
    g_ref[1, :, :] = hw[:, 128:]


def _k2(p, dis, b1, W1, W2):
    grid = NPAD // BR
    return pl.pallas_call(
        _k2_body,
        grid=(grid,),
        in_specs=[
            pl.BlockSpec((NC, BR, 128), lambda i: (0, i, 0)),
            pl.BlockSpec((BR, 1), lambda i: (i, 0)),
            pl.BlockSpec((1, 256), lambda i: (0, 0)),
            pl.BlockSpec((128, 256), lambda i: (0, 0)),
            pl.BlockSpec((256, 256), lambda i: (0, 0)),
        ],
        out_specs=pl.BlockSpec((NC, BR, 128), lambda i: (0, i, 0)),
        out_shape=jax.ShapeDtypeStruct((NC, NPAD, 128), jnp.float32),
    )(p, dis, b1, W1, W2)


def _mk_mid_body(split_out):
    def body(a_ref, dis_ref, b_ref, w_ref, g_ref):
        dis = dis_ref[...]
        s0 = jax.nn.relu(a_ref[0, :, :] * dis + b_ref[:, :128])
        s1 = jax.nn.relu(a_ref[1, :, :] * dis + b_ref[:, 128:])
        hw = (jnp.dot(s0, w_ref[:128, :],
                      preferred_element_type=jnp.float32,
                      precision=lax.Precision.HIGHEST)
              + jnp.dot(s1, w_ref[128:, :],
                        preferred_element_type=jnp.float32,
                        precision=lax.Precision.HIGHEST)) * dis
        if split_out:
            g_ref[0, :, :] = hw[:, :128]
            g_ref[1, :, :] = hw[:, 128:]
        else:
            g_ref[...] = hw
    return body


def _mid(a, dis, b, W, split_out):
    grid = NPAD // BR
    dout = W.shape[1]
    if split_out:
        out_spec = pl.BlockSpec((NC, BR, 128), lambda i: (0, i, 0))
        out_shape = jax.ShapeDtypeStruct((NC, NPAD, 128), jnp.float32)
    else:
        out_spec = pl.BlockSpec((BR, dout), lambda i: (i, 0))
        out_shape = jax.ShapeDtypeStruct((NPAD, dout), jnp.float32)
    return pl.pallas_call(
        _mk_mid_body(split_out),
        grid=(grid,),
        in_specs=[
            pl.BlockSpec((NC, BR, 128), lambda i: (0, i, 0)),
            pl.BlockSpec((BR, 1), lambda i: (i, 0)),
            pl.BlockSpec((1, 256), lambda i: (0, 0)),
            pl.BlockSpec((256, dout), lambda i: (0, 0)),
        ],
        out_specs=out_spec,
        out_shape=out_shape,
    )(a, dis, b, W)


def _k4_body(a_ref, dis_ref, b_ref, o_ref):
    a = a_ref[0, :, :] + a_ref[1, :, :]
    o_ref[...] = jax.nn.relu(a * dis_ref[...] + b_ref[...])


def _k4(a, dis, b3):
    grid = NPAD // BR
    return pl.pallas_call(
        _k4_body,
        grid=(grid,),
        in_specs=[
            pl.BlockSpec((NC, BR, 128), lambda i: (0, i, 0)),
            pl.BlockSpec((BR, 1), lambda i: (i, 0)),
            pl.BlockSpec((1, 128), lambda i: (0, 0)),
        ],
        out_specs=pl.BlockSpec((BR, 128), lambda i: (i, 0)),
        out_shape=jax.ShapeDtypeStruct((NPAD, 128), jnp.float32),
    )(a, dis, b3)


def _pad_edges(src, dst, total):
    npad = total - src.shape[0]
    pad_src = (jnp.arange(npad, dtype=jnp.int32) * 181) % N
    pad_dst = N + (jnp.arange(npad, dtype=jnp.int32) % (NPAD - N))
    return (jnp.concatenate([src, pad_src]), jnp.concatenate([dst, pad_dst]))


def kernel(x, batch_edge_index, W1, b1, W2, b2, W3, b3):
    n_batch, n_points, _ = x.shape
    h0 = jnp.pad(x.reshape(-1, x.shape[-1]), ((0, NPAD - N), (0, 0)))

    loop = jnp.arange(N, dtype=jnp.int32)
    src = jnp.concatenate([batch_edge_index[0], loop])
    dst = jnp.concatenate([batch_edge_index[1], loop])

    srcc, dstc = _pad_edges(src, dst, E_COL)
    src_col = srcc.reshape(NS, T, B)
    src_off = jnp.stack([src_col, src_col + NPAD])
    dst_col = dstc.reshape(NS, T, B)

    src_edge = srcc.reshape(NC * NS, TER, BE)
    dst_edge = dstc.reshape(NC * NS, TER, BE)

    zeros128 = jnp.zeros((RPT, 128), jnp.float32)
    ones_b = jnp.ones((BE, 128), jnp.float32)

    deg_parts = _deg_scatter(dst_edge, zeros128, ones_b)
    dis, z = _k1(h0, deg_parts)
    p1 = _scatter_edge(z, src_edge, dst_edge, zeros128)
    g2 = _k2(p1, dis, b1.reshape(1, 256), W1, W2)
    a2 = _scatter_col(g2.reshape(NC * NPAD, 128), src_off, dst_col, zeros128)
    g3 = _mid(a2, dis, b2.reshape(1, 256), W3, split_out=False)
    a3 = _scatter_edge(g3, src_edge, dst_edge, zeros128)
    out = _k4(a3, dis, b3.reshape(1, 128))
    return out[:N].reshape(n_batch, n_points, -1)
